# baseline (speedup 1.0000x reference)
import jax
import jax.numpy as jnp
from jax import lax
from jax.experimental import pallas as pl
from jax.experimental.pallas import tpu as pltpu

_L = 2048
_T = 32
_D = 32
_G = _L // _T
_SB = 4
_RW = _SB * _D


def _body(conv_ref, hWT_ref, hb_ref, wBD_ref, wb_ref, a_any, v_any,
          oa_ref, ov_ref, a_buf, v_buf, sems):
    i = pl.program_id(0)
    n = pl.num_programs(0)
    f32 = jnp.float32

    def start(step, slot):
        pltpu.make_async_copy(
            a_any.at[pl.ds(step * _SB, _SB)], a_buf.at[slot],
            sems.at[slot, 0]).start()
        pltpu.make_async_copy(
            v_any.at[pl.ds(step * _SB, _SB)], v_buf.at[slot],
            sems.at[slot, 1]).start()

    @pl.when(i == 0)
    def _():
        start(0, 0)

    @pl.when(i + 1 < n)
    def _():
        start(i + 1, (i + 1) % 2)

    slot = i % 2
    pltpu.make_async_copy(
        a_any.at[pl.ds(i * _SB, _SB)], a_buf.at[slot], sems.at[slot, 0]).wait()
    pltpu.make_async_copy(
        v_any.at[pl.ds(i * _SB, _SB)], v_buf.at[slot], sems.at[slot, 1]).wait()

    a = a_buf[slot].reshape(_RW, _L)
    v = v_buf[slot].reshape(_RW, _L)
    ti = lax.broadcasted_iota(jnp.int32, (_L, _T), 0) // _G
    gi = lax.broadcasted_iota(jnp.int32, (_L, _T), 1)
    Q = (ti == gi).astype(f32)
    A_r = jnp.dot(a, Q) * (1.0 / _G)
    V_r = jnp.dot(v, Q) * (1.0 / _G)
    w0 = conv_ref[0]
    w1 = conv_ref[1]
    cb = conv_ref[2]
    c = jax.nn.sigmoid(w0 * A_r + w1 * V_r + cb)
    hw = (A_r + V_r) * 0.5
    si = lax.broadcasted_iota(jnp.int32, (_SB, _RW), 0)
    ri = lax.broadcasted_iota(jnp.int32, (_SB, _RW), 1) // _D
    E = (si == ri).astype(f32) * (1.0 / _D)
    rm = jnp.dot(E, hw)
    H = jax.nn.sigmoid(jnp.dot(rm, hWT_ref[...]) + hb_ref[...])
    cm = jnp.mean(hw, axis=1, keepdims=True)
    w = jax.nn.sigmoid(jnp.dot(wBD_ref[...], cm) + wb_ref[...])
    MT = (E > 0.0).astype(f32)
    dn_bc = (((0,), (0,)), ((), ()))
    Hb = lax.dot_general(MT, H, dn_bc)
    S = (Hb + w + c) * (1.0 / 3.0)
    ug = lax.broadcasted_iota(jnp.int32, (_T, _L), 0)
    ut = lax.broadcasted_iota(jnp.int32, (_T, _L), 1) // _G
    U = (ug == ut).astype(f32)
    scale = jnp.dot(S, U)
    oa_ref[...] = (a * scale).reshape(_SB, _D, _L)
    ov_ref[...] = (v * scale).reshape(_SB, _D, _L)


def kernel(acoustic_seq, visual_seq, IS_BAG_list, hW, hb, wW, wb, convW,
           convb):
    del IS_BAG_list
    B = acoustic_seq.shape[0]
    at = jnp.transpose(acoustic_seq, (0, 2, 1))
    vt = jnp.transpose(visual_seq, (0, 2, 1))
    conv = jnp.stack([convW[0, 0, 0, 0], convW[0, 1, 0, 0], convb[0]])
    hWT = hW.T
    hb2 = hb.reshape(1, _T)
    wBD = jax.scipy.linalg.block_diag(*([wW] * _SB))
    wb4 = jnp.tile(wb, _SB).reshape(_RW, 1)
    seq_spec = pl.BlockSpec((_SB, _D, _L), lambda i: (i, 0, 0))
    full = lambda *s: pl.BlockSpec(s, lambda i: tuple(0 for _ in s))
    out_a, out_v = pl.pallas_call(
        _body,
        grid=(B // _SB,),
        in_specs=[
            pl.BlockSpec(memory_space=pltpu.SMEM),
            full(_T, _T),
            full(1, _T),
            full(_RW, _RW),
            full(_RW, 1),
            pl.BlockSpec(memory_space=pl.ANY),
            pl.BlockSpec(memory_space=pl.ANY),
        ],
        out_specs=[seq_spec, seq_spec],
        out_shape=[
            jax.ShapeDtypeStruct((B, _D, _L), jnp.float32),
            jax.ShapeDtypeStruct((B, _D, _L), jnp.float32),
        ],
        scratch_shapes=[
            pltpu.VMEM((2, _SB, _D, _L), jnp.float32),
            pltpu.VMEM((2, _SB, _D, _L), jnp.float32),
            pltpu.SemaphoreType.DMA((2, 2)),
        ],
    )(conv, hWT, hb2, wBD, wb4, at, vt)
    return jnp.transpose(out_a, (0, 2, 1)), jnp.transpose(out_v, (0, 2, 1))
